# baseline (speedup 1.0000x reference)
import functools

import jax
import jax.numpy as jnp
from jax import lax
from jax.experimental import pallas as pl
from jax.experimental.pallas import tpu as pltpu
from jax.experimental.pallas import tpu_sc as plsc

NC = 2
NS = 16
NW = NC * NS
LANES = 16


def _make_sc_lookup(B, L, V, P, D):
    rows_per_w = B // NW
    split = min(128, L)

    def body(x_hbm, tok_hbm, pos_hbm, out_hbm,
             pos_v, idx0, idx1, out0, out1, sem0, sem1):
        wid = lax.axis_index("s") * NC + lax.axis_index("c")
        base = wid * rows_per_w

        pltpu.sync_copy(pos_hbm.at[pl.ds(0, L)], pos_v)

        idx = (idx0, idx1)
        outv = (out0, out1)
        sem = (sem0, sem1)

        def stage(b, s):
            pltpu.sync_copy(x_hbm.at[base + b], idx[s])
            pltpu.async_copy(tok_hbm.at[idx[s].at[pl.ds(0, split)]],
                             outv[s].at[pl.ds(0, split)], sem[s])
            if L > split:
                pltpu.async_copy(tok_hbm.at[idx[s].at[pl.ds(split, L - split)]],
                                 outv[s].at[pl.ds(split, L - split)], sem[s])

        def finish(b, s):
            pltpu.make_async_copy(tok_hbm.at[idx[s]], outv[s], sem[s]).wait()

            def add_rows(j, carry):
                r = j * 4
                for rr in range(4):
                    for c in range(D // LANES):
                        plsc.addupdate(
                            outv[s].at[r + rr, pl.ds(c * LANES, LANES)],
                            pos_v[r + rr, pl.ds(c * LANES, LANES)])
                return carry

            lax.fori_loop(0, L // 4, add_rows, 0)
            pltpu.sync_copy(outv[s], out_hbm.at[base + b])

        stage(0, 0)

        def step(i, carry):
            bb = i * 2
            stage(bb + 1, 1)
            finish(bb, 0)

            @pl.when(bb + 2 < rows_per_w)
            def _():
                stage(bb + 2, 0)

            finish(bb + 1, 1)
            return carry

        lax.fori_loop(0, rows_per_w // 2, step, 0)

    mesh = plsc.VectorSubcoreMesh(core_axis_name="c", subcore_axis_name="s",
                                  num_cores=NC, num_subcores=NS)
    return pl.kernel(
        body,
        out_type=jax.ShapeDtypeStruct((B, L, D), jnp.float32),
        mesh=mesh,
        scratch_types=[
            pltpu.VMEM((L, D), jnp.float32),
            pltpu.VMEM((L,), jnp.int32),
            pltpu.VMEM((L,), jnp.int32),
            pltpu.VMEM((L, D), jnp.float32),
            pltpu.VMEM((L, D), jnp.float32),
            pltpu.SemaphoreType.DMA,
            pltpu.SemaphoreType.DMA,
        ],
        compiler_params=pltpu.CompilerParams(use_tc_tiling_on_sc=False),
        name="sc_embedding_lookup",
    )


def kernel(x, token_emb, pos_emb):
    B, L = x.shape
    V, D = token_emb.shape
    P = pos_emb.shape[0]
    fn = _make_sc_lookup(B, L, V, P, D)
    return fn(x, token_emb, pos_emb)


# speedup vs baseline: 1.0832x; 1.0832x over previous
import jax
import jax.numpy as jnp
from jax import lax
from jax.experimental import pallas as pl
from jax.experimental.pallas import tpu as pltpu
from jax.experimental.pallas import tpu_sc as plsc

NC = 2
NS = 16
NW = NC * NS
LANES = 16
NBUF = 3


def _make_sc_lookup(B, L, D):
    n_flat = B * L
    per_w = n_flat // NW
    G = 2 * L
    n_it = per_w // G
    chunks = []
    o = 0
    while o < G:
        n = min(128, G - o)
        chunks.append((o, n))
        o += n

    def body(x_hbm, tok_hbm, pos_hbm, out_hbm,
             pos_v, idx_all, o0, o1, o2, g0, g1, g2, s0, s1, s2):
        wid = lax.axis_index("s") * NC + lax.axis_index("c")
        slab = wid * per_w

        outs = (o0, o1, o2)
        gsem = (g0, g1, g2)
        ssem = (s0, s1, s2)

        pltpu.sync_copy(pos_hbm.at[pl.ds(0, L)], pos_v)
        pltpu.sync_copy(x_hbm.at[pl.ds(slab, per_w)], idx_all)

        def stage(it, s, drain):
            if drain:
                pltpu.make_async_copy(
                    outs[s], out_hbm.at[pl.ds(slab, G)], ssem[s]).wait()
            for (co, cn) in chunks:
                pltpu.async_copy(
                    tok_hbm.at[idx_all.at[pl.ds(it * G + co, cn)]],
                    outs[s].at[pl.ds(co, cn)], gsem[s])

        def finish(it, s):
            for (co, cn) in chunks:
                pltpu.make_async_copy(
                    tok_hbm.at[idx_all.at[pl.ds(it * G + co, cn)]],
                    outs[s].at[pl.ds(co, cn)], gsem[s]).wait()

            def add_rows(j, carry):
                r = j * 4
                for rr in range(4):
                    for c in range(D // LANES):
                        p = pos_v[r + rr, pl.ds(c * LANES, LANES)]
                        plsc.addupdate(
                            outs[s].at[r + rr, pl.ds(c * LANES, LANES)], p)
                        plsc.addupdate(
                            outs[s].at[L + r + rr, pl.ds(c * LANES, LANES)], p)
                return carry

            lax.fori_loop(0, L // 4, add_rows, 0)
            pltpu.async_copy(
                outs[s], out_hbm.at[pl.ds(slab + it * G, G)], ssem[s])

        for s in range(NBUF):
            stage(s, s, drain=False)

        n_main = (n_it - NBUF) // NBUF

        def step(i, carry):
            for s in range(NBUF):
                it = i * NBUF + s
                finish(it, s)
                stage(it + NBUF, s, drain=True)
            return carry

        lax.fori_loop(0, n_main, step, 0)

        for it in range(n_main * NBUF, n_it):
            s = it % NBUF
            finish(it, s)
            if it + NBUF < n_it:
                stage(it + NBUF, s, drain=True)

        for s in range(NBUF):
            pltpu.make_async_copy(
                outs[s], out_hbm.at[pl.ds(slab, G)], ssem[s]).wait()

    mesh = plsc.VectorSubcoreMesh(core_axis_name="c", subcore_axis_name="s",
                                  num_cores=NC, num_subcores=NS)
    return pl.kernel(
        body,
        out_type=jax.ShapeDtypeStruct((n_flat, D), jnp.float32),
        mesh=mesh,
        scratch_types=[
            pltpu.VMEM((L, D), jnp.float32),
            pltpu.VMEM((per_w,), jnp.int32),
            pltpu.VMEM((G, D), jnp.float32),
            pltpu.VMEM((G, D), jnp.float32),
            pltpu.VMEM((G, D), jnp.float32),
            pltpu.SemaphoreType.DMA,
            pltpu.SemaphoreType.DMA,
            pltpu.SemaphoreType.DMA,
            pltpu.SemaphoreType.DMA,
            pltpu.SemaphoreType.DMA,
            pltpu.SemaphoreType.DMA,
        ],
        compiler_params=pltpu.CompilerParams(use_tc_tiling_on_sc=False),
        name="sc_embedding_lookup",
    )


def kernel(x, token_emb, pos_emb):
    B, L = x.shape
    _, D = token_emb.shape
    fn = _make_sc_lookup(B, L, D)
    out = fn(x.reshape(-1), token_emb, pos_emb)
    return out.reshape(B, L, D)


# speedup vs baseline: 1.1874x; 1.0962x over previous
import jax
import jax.numpy as jnp
from jax import lax
from jax.experimental import pallas as pl
from jax.experimental.pallas import tpu as pltpu
from jax.experimental.pallas import tpu_sc as plsc

NC = 2
NS = 16
NW = NC * NS
LANES = 16
TSTRIDE = 129


def _make_sc_lookup(B, L, D):
    assert B % (128 * NW // NW) == 0 and B // 128 == NW
    DT = D // 8
    DC = D // LANES

    def body(xt_hbm, tok_hbm, pos_hbm, out_hbm,
             pos_v, idx_all, g0, g1, g2, t0, t1,
             gs0, gs1, gs2, ss0, ss1):
        w = lax.axis_index("s") * NC + lax.axis_index("c")

        G = (g0, g1, g2)
        T = (t0, t1)
        gsem = (gs0, gs1, gs2)
        ssem = (ss0, ss1)

        pltpu.sync_copy(pos_hbm, pos_v)
        pltpu.sync_copy(xt_hbm.at[:, pl.ds(w * 128, 128)], idx_all)

        row_idx = [jnp.arange(LANES, dtype=jnp.int32) + dc * LANES
                   for dc in range(DC)]

        def stage(l, gs):
            pltpu.async_copy(tok_hbm.at[idx_all.at[l]], G[gs], gsem[gs])

        def finish(l, gs, ts, drain):
            pltpu.make_async_copy(
                tok_hbm.at[idx_all.at[l]], G[gs], gsem[gs]).wait()
            if drain:
                for dt in range(DT):
                    pltpu.make_async_copy(
                        T[ts].at[pl.ds(dt * 8, 8), pl.ds(0, 128)],
                        out_hbm.at[l, dt, w], ssem[ts]).wait()

            pos_c = [pos_v[l, pl.ds(dc * LANES, LANES)] for dc in range(DC)]

            def tbody(t, carry):
                col = jnp.full((LANES,), t, dtype=jnp.int32)
                for dc in range(DC):
                    v = G[gs][t, pl.ds(dc * LANES, LANES)] + pos_c[dc]
                    plsc.store_scatter(T[ts], [row_idx[dc], col], v)
                return carry

            lax.fori_loop(0, 128, tbody, 0)

            for dt in range(DT):
                pltpu.async_copy(
                    T[ts].at[pl.ds(dt * 8, 8), pl.ds(0, 128)],
                    out_hbm.at[l, dt, w], ssem[ts])

        for l in range(3):
            stage(l, l % 3)
        finish(0, 0, 0, drain=False)
        stage(3, 0)
        finish(1, 1, 1, drain=False)
        stage(4, 1)

        def step(i, carry):
            for k in range(6):
                l = 2 + i * 6 + k
                finish(l, (2 + k) % 3, k % 2, drain=True)
                stage(l + 3, (2 + k) % 3)
            return carry

        lax.fori_loop(0, (L - 8) // 6, step, 0)

        for l in range(L - 6, L):
            finish(l, l % 3, l % 2, drain=True)
            if l + 3 < L:
                stage(l + 3, l % 3)

        for ts in range(2):
            for dt in range(DT):
                pltpu.make_async_copy(
                    T[ts].at[pl.ds(dt * 8, 8), pl.ds(0, 128)],
                    out_hbm.at[L - 1, dt, w], ssem[ts]).wait()

    mesh = plsc.VectorSubcoreMesh(core_axis_name="c", subcore_axis_name="s",
                                  num_cores=NC, num_subcores=NS)
    return pl.kernel(
        body,
        out_type=jax.ShapeDtypeStruct((L, DT, NW, 8, 128), jnp.float32),
        mesh=mesh,
        scratch_types=[
            pltpu.VMEM((L, D), jnp.float32),
            pltpu.VMEM((L, 128), jnp.int32),
            pltpu.VMEM((128, D), jnp.float32),
            pltpu.VMEM((128, D), jnp.float32),
            pltpu.VMEM((128, D), jnp.float32),
            pltpu.VMEM((D, TSTRIDE), jnp.float32),
            pltpu.VMEM((D, TSTRIDE), jnp.float32),
            pltpu.SemaphoreType.DMA,
            pltpu.SemaphoreType.DMA,
            pltpu.SemaphoreType.DMA,
            pltpu.SemaphoreType.DMA,
            pltpu.SemaphoreType.DMA,
        ],
        compiler_params=pltpu.CompilerParams(use_tc_tiling_on_sc=False,
                                             needs_layout_passes=False),
        name="sc_embedding_lookup",
    )


def kernel(x, token_emb, pos_emb):
    B, L = x.shape
    _, D = token_emb.shape
    fn = _make_sc_lookup(B, L, D)
    out5d = fn(x.T, token_emb, pos_emb[:L])
    return out5d.transpose(2, 4, 0, 1, 3).reshape(B, L, D)
